# baseline (speedup 1.0000x reference)
import functools
import math

import jax
import jax.numpy as jnp
from jax import lax
from jax.experimental import pallas as pl
from jax.experimental.pallas import tpu as pltpu
from jax.experimental.pallas import tpu_sc as plsc

D_MODEL = 512
SCALE = math.sqrt(D_MODEL)
LANES = 16

NUM_CORES = 2
NUM_SUBCORES = 16
NW = NUM_CORES * NUM_SUBCORES

SEQ_PAD = 56
NBUF = 2


def _emb_body(rows_per_w, table_hbm, idx_hbm, out_hbm,
              idx_v, bufs, gsems, ssems):
    wid = lax.axis_index("s") * NUM_CORES + lax.axis_index("c")
    xbase = wid * rows_per_w

    pltpu.sync_copy(idx_hbm.at[wid], idx_v)

    def gather(r, b):
        pltpu.async_copy(table_hbm.at[idx_v.at[r]], bufs[b], gsems[b])

    def wait_gather(r, b):
        pltpu.make_async_copy(
            table_hbm.at[idx_v.at[r]], bufs[b], gsems[b]).wait()

    def store(r, b):
        pltpu.async_copy(bufs[b], out_hbm.at[xbase + r], ssems[b])

    def wait_store(r, b):
        pltpu.make_async_copy(bufs[b], out_hbm.at[xbase + r], ssems[b]).wait()

    gather(0, 0)

    def group_body(r0, _):
        for b in range(NBUF):
            r = r0 + b
            b2 = (b + 1) % NBUF
            wait_gather(r, b)

            @pl.when(r + 1 < rows_per_w)
            def _():
                @pl.when(r >= 1)
                def _():
                    wait_store(r - 1, b2)
                gather(r + 1, b2)

            @plsc.parallel_loop(0, SEQ_PAD, step=1, unroll=2)
            def _(i):
                for j in range(D_MODEL // LANES):
                    sl = pl.ds(j * LANES, LANES)
                    bufs[b][i, sl] = bufs[b][i, sl] * SCALE

            store(r, b)
        return 0

    lax.fori_loop(0, rows_per_w // NBUF,
                  lambda i, a: group_body(i * NBUF, a), 0)

    for b in range(NBUF):
        wait_store(rows_per_w - NBUF + b, b)


@functools.lru_cache(maxsize=None)
def _make_emb(nrows):
    assert nrows % NW == 0
    rows_per_w = nrows // NW
    assert rows_per_w % NBUF == 0
    mesh = plsc.VectorSubcoreMesh(
        core_axis_name="c", subcore_axis_name="s",
        num_cores=NUM_CORES, num_subcores=NUM_SUBCORES)
    return pl.kernel(
        functools.partial(_emb_body, rows_per_w),
        out_type=jax.ShapeDtypeStruct((nrows, SEQ_PAD, D_MODEL), jnp.float32),
        mesh=mesh,
        scratch_types=[
            pltpu.VMEM((rows_per_w, SEQ_PAD), jnp.int32),
            [pltpu.VMEM((SEQ_PAD, D_MODEL), jnp.float32)
             for _ in range(NBUF)],
            [pltpu.SemaphoreType.DMA for _ in range(NBUF)],
            [pltpu.SemaphoreType.DMA for _ in range(NBUF)],
        ],
    )


def kernel(x, table):
    nrows, seq = x.shape
    idx = jnp.pad(x.astype(jnp.int32), ((0, 0), (0, SEQ_PAD - seq)))
    idx = idx.reshape(NW, nrows // NW, SEQ_PAD)
    out = _make_emb(nrows)(table, idx)
    return out[:, :seq, :]


# speedup vs baseline: 6.5546x; 6.5546x over previous
import functools
import math

import jax
import jax.numpy as jnp
from jax import lax
from jax.experimental import pallas as pl
from jax.experimental.pallas import tpu as pltpu
from jax.experimental.pallas import tpu_sc as plsc

D_MODEL = 512
SCALE = math.sqrt(D_MODEL)
LANES = 16

NUM_CORES = 2
NUM_SUBCORES = 16
NW = NUM_CORES * NUM_SUBCORES

CHUNK = 64
NBUF = 2


def _emb_body(seq, nrows, table_hbm, idx_hbm, out_hbm,
              idx_v, bufs, gsems, ssems):
    wid = lax.axis_index("s") * NUM_CORES + lax.axis_index("c")
    n_per_w = nrows // NW
    halves = n_per_w // CHUNK
    nchunks = seq * halves
    nbase = wid * n_per_w

    pltpu.sync_copy(idx_hbm.at[wid], idx_v)

    def dst(c):
        s = c // halves
        h = c % halves
        return out_hbm.at[s, pl.ds(nbase + h * CHUNK, CHUNK)]

    def gather(c, b):
        pltpu.async_copy(table_hbm.at[idx_v.at[c]], bufs[b], gsems[b])

    def wait_gather(c, b):
        pltpu.make_async_copy(
            table_hbm.at[idx_v.at[c]], bufs[b], gsems[b]).wait()

    def store(c, b):
        pltpu.async_copy(bufs[b], dst(c), ssems[b])

    def wait_store(c, b):
        pltpu.make_async_copy(bufs[b], dst(c), ssems[b]).wait()

    gather(0, 0)

    def group_body(c0, _):
        for b in range(NBUF):
            c = c0 + b
            b2 = (b + 1) % NBUF
            wait_gather(c, b)

            @pl.when(c + 1 < nchunks)
            def _():
                @pl.when(c >= 1)
                def _():
                    wait_store(c - 1, b2)
                gather(c + 1, b2)

            @plsc.parallel_loop(0, CHUNK, step=1, unroll=2)
            def _(i):
                for j in range(D_MODEL // LANES):
                    sl = pl.ds(j * LANES, LANES)
                    bufs[b][i, sl] = bufs[b][i, sl] * SCALE

            store(c, b)
        return 0

    lax.fori_loop(0, nchunks // NBUF,
                  lambda i, a: group_body(i * NBUF, a), 0)

    for b in range(NBUF):
        wait_store(nchunks - NBUF + b, b)


@functools.lru_cache(maxsize=None)
def _make_emb(nrows, seq):
    assert nrows % (NW * CHUNK) == 0
    halves = nrows // NW // CHUNK
    nchunks = seq * halves
    assert nchunks % NBUF == 0
    mesh = plsc.VectorSubcoreMesh(
        core_axis_name="c", subcore_axis_name="s",
        num_cores=NUM_CORES, num_subcores=NUM_SUBCORES)
    return pl.kernel(
        functools.partial(_emb_body, seq, nrows),
        out_type=jax.ShapeDtypeStruct((seq, nrows, D_MODEL), jnp.float32),
        mesh=mesh,
        scratch_types=[
            pltpu.VMEM((nchunks, CHUNK), jnp.int32),
            [pltpu.VMEM((CHUNK, D_MODEL), jnp.float32) for _ in range(NBUF)],
            [pltpu.SemaphoreType.DMA for _ in range(NBUF)],
            [pltpu.SemaphoreType.DMA for _ in range(NBUF)],
        ],
    )


def kernel(x, table):
    nrows, seq = x.shape
    halves = nrows // NW // CHUNK
    idx = (x.astype(jnp.int32).T
           .reshape(seq, NW, halves, CHUNK)
           .transpose(1, 0, 2, 3)
           .reshape(NW, seq * halves, CHUNK))
    out_t = _make_emb(nrows, seq)(table, idx)
    return out_t.transpose(1, 0, 2)


# speedup vs baseline: 6.5600x; 1.0008x over previous
import functools
import math

import jax
import jax.numpy as jnp
from jax import lax
from jax.experimental import pallas as pl
from jax.experimental.pallas import tpu as pltpu
from jax.experimental.pallas import tpu_sc as plsc

D_MODEL = 512
SCALE = math.sqrt(D_MODEL)
LANES = 16

NUM_CORES = 2
NUM_SUBCORES = 16
NW = NUM_CORES * NUM_SUBCORES

CHUNK = 64
NBUF = 3


def _emb_body(seq, nrows, table_hbm, idx_hbm, out_hbm,
              idx_v, bufs, gsems, ssems):
    wid = lax.axis_index("s") * NUM_CORES + lax.axis_index("c")
    n_per_w = nrows // NW
    halves = n_per_w // CHUNK
    nchunks = seq * halves
    nbase = wid * n_per_w

    pltpu.sync_copy(idx_hbm.at[wid], idx_v)

    def dst(c):
        s = c // halves
        h = c % halves
        return out_hbm.at[s, pl.ds(nbase + h * CHUNK, CHUNK)]

    def gather(c, b):
        pltpu.async_copy(table_hbm.at[idx_v.at[c]], bufs[b], gsems[b])

    def wait_gather(c, b):
        pltpu.make_async_copy(
            table_hbm.at[idx_v.at[c]], bufs[b], gsems[b]).wait()

    def store(c, b):
        pltpu.async_copy(bufs[b], dst(c), ssems[b])

    def wait_store(c, b):
        pltpu.make_async_copy(bufs[b], dst(c), ssems[b]).wait()

    def scale(b):
        @plsc.parallel_loop(0, CHUNK, step=1, unroll=2)
        def _(i):
            for j in range(D_MODEL // LANES):
                sl = pl.ds(j * LANES, LANES)
                bufs[b][i, sl] = bufs[b][i, sl] * SCALE

    def chunk_step(c, b):
        wait_gather(c, b)
        b2 = (b + NBUF - 1) % NBUF

        @pl.when(c + NBUF - 1 < nchunks)
        def _():
            @pl.when(c >= 1)
            def _():
                wait_store(c - 1, b2)
            gather(c + NBUF - 1, b2)

        scale(b)
        store(c, b)

    for b in range(NBUF - 1):
        gather(b, b)

    ngroups = nchunks // NBUF

    def group_body(c0, _):
        for b in range(NBUF):
            chunk_step(c0 + b, b)
        return 0

    lax.fori_loop(0, ngroups, lambda i, a: group_body(i * NBUF, a), 0)

    for j in range(nchunks % NBUF):
        c = ngroups * NBUF + j
        b = c % NBUF
        wait_gather(c, b)
        scale(b)
        store(c, b)

    for c in range(nchunks - NBUF, nchunks):
        wait_store(c, c % NBUF)


@functools.lru_cache(maxsize=None)
def _make_emb(nrows, seq):
    assert nrows % (NW * CHUNK) == 0
    halves = nrows // NW // CHUNK
    nchunks = seq * halves
    assert nchunks >= NBUF
    mesh = plsc.VectorSubcoreMesh(
        core_axis_name="c", subcore_axis_name="s",
        num_cores=NUM_CORES, num_subcores=NUM_SUBCORES)
    return pl.kernel(
        functools.partial(_emb_body, seq, nrows),
        out_type=jax.ShapeDtypeStruct((seq, nrows, D_MODEL), jnp.float32),
        mesh=mesh,
        scratch_types=[
            pltpu.VMEM((nchunks, CHUNK), jnp.int32),
            [pltpu.VMEM((CHUNK, D_MODEL), jnp.float32) for _ in range(NBUF)],
            [pltpu.SemaphoreType.DMA for _ in range(NBUF)],
            [pltpu.SemaphoreType.DMA for _ in range(NBUF)],
        ],
    )


def kernel(x, table):
    nrows, seq = x.shape
    halves = nrows // NW // CHUNK
    idx = (x.astype(jnp.int32).T
           .reshape(seq, NW, halves, CHUNK)
           .transpose(1, 0, 2, 3)
           .reshape(NW, seq * halves, CHUNK))
    out_t = _make_emb(nrows, seq)(table, idx)
    return out_t.transpose(1, 0, 2)
